# baseline (speedup 1.0000x reference)
import functools

import jax
import jax.numpy as jnp
from jax import lax
from jax.experimental import pallas as pl
from jax.experimental.pallas import tpu as pltpu
from jax.experimental.pallas import tpu_sc as plsc

LATENT = 128
N_NODES = 10000
N_EDGES = 320000
N_GRAPHS = 64
MAX_LV = 3

NC, NS = 2, 16
HALF = LATENT // NC
CHUNK = 128
N_CHUNKS = 160
E_PAD = NS * N_CHUNKS * CHUNK
NBUF = 4
ROWS_PER_SUB = N_NODES // NS
ZROWS = 25
JUNK = N_NODES
N_JUNK = 128
PAD_PER_SUB = (E_PAD - N_EDGES) // NS
E_PER_SUB = N_EDGES // NS

BLK = 2000
N_BLK = N_NODES // BLK


def _spmm_body(cur_hbm, src_hbm, dst_hbm, agg_hbm,
               agg_sh, src_v, dst_v, rows, zbuf, sems):
    c = lax.axis_index("c")
    s = lax.axis_index("s")

    pltpu.sync_copy(src_hbm.at[s], src_v)
    pltpu.sync_copy(dst_hbm.at[s], dst_v)

    zero = jnp.zeros((16,), jnp.float32)
    for r in range(ZROWS):
        for q in range(HALF // 16):
            zbuf[r, pl.ds(q * 16, 16)] = zero
    base_row = s * ROWS_PER_SUB

    def zbody(i, carry):
        pltpu.sync_copy(zbuf, agg_sh.at[pl.ds(base_row + i * ZROWS, ZROWS)])
        return carry

    lax.fori_loop(0, ROWS_PER_SUB // ZROWS, zbody, 0)
    pltpu.sync_copy(zbuf.at[pl.ds(0, N_JUNK // NS)],
                    agg_sh.at[pl.ds(N_NODES + s * (N_JUNK // NS), N_JUNK // NS)])
    plsc.subcore_barrier()

    def cbody(r, carry):
        for q in range(CHUNK // 16):
            src_v[r, pl.ds(q * 16, 16)] = src_v[r, pl.ds(q * 16, 16)] + c
        return carry

    lax.fori_loop(0, N_CHUNKS, cbody, 0)

    cur_c = cur_hbm

    def start(j, q):
        pltpu.async_copy(cur_c.at[src_v.at[j]], rows[q], sems[q])

    def finish(j, q):
        pltpu.make_async_copy(cur_c.at[src_v.at[j]], rows[q], sems[q]).wait()
        pltpu.sync_copy(rows[q], agg_sh.at[dst_v.at[j]], add=True)

    for q in range(NBUF - 1):
        start(q, q)

    def body(t, carry):
        j0 = NBUF * t
        for q in range(NBUF):
            finish(j0 + q, q)
            start(j0 + q + NBUF - 1, (q + NBUF - 1) % NBUF)
        return carry

    lax.fori_loop(0, N_CHUNKS // NBUF - 1, body, 0)
    j0 = N_CHUNKS - NBUF
    finish(j0, 0)
    start(N_CHUNKS - 1, NBUF - 1)
    for q in range(1, NBUF):
        finish(j0 + q, q)

    plsc.subcore_barrier()
    pltpu.sync_copy(agg_sh.at[pl.ds(base_row, ROWS_PER_SUB)],
                    agg_hbm.at[pl.ds(base_row, ROWS_PER_SUB),
                               pl.ds(c * HALF, HALF)])


@functools.lru_cache(maxsize=1)
def _get_spmm():
    return pl.kernel(
        _spmm_body,
        out_type=jax.ShapeDtypeStruct((N_NODES, LATENT), jnp.float32),
        mesh=plsc.VectorSubcoreMesh(core_axis_name="c", subcore_axis_name="s",
                                    num_cores=NC, num_subcores=NS),
        scratch_types=[
            pltpu.VMEM_SHARED((N_NODES + N_JUNK, HALF), jnp.float32),
            pltpu.VMEM((N_CHUNKS, CHUNK), jnp.int32),
            pltpu.VMEM((N_CHUNKS, CHUNK), jnp.int32),
            [pltpu.VMEM((CHUNK, HALF), jnp.float32) for _ in range(NBUF)],
            pltpu.VMEM((ZROWS, HALF), jnp.float32),
            [pltpu.SemaphoreType.DMA for _ in range(NBUF)],
        ],
        compiler_params=pltpu.CompilerParams(use_tc_tiling_on_sc=False),
    )


def _spmm(cur, src2, dst):
    return _get_spmm()(jnp.reshape(cur, (2 * N_NODES, HALF)), src2, dst)


def _proj_body(x_ref, w_ref, im_ref, cur_ref):
    im = jnp.dot(x_ref[...], w_ref[...], preferred_element_type=jnp.float32)
    im_ref[...] = im
    cur_ref[...] = jnp.maximum(im, 0.0)


def _proj(x, w):
    return pl.pallas_call(
        _proj_body,
        grid=(N_BLK,),
        in_specs=[
            pl.BlockSpec((BLK, LATENT), lambda i: (i, 0)),
            pl.BlockSpec((LATENT, LATENT), lambda i: (0, 0)),
        ],
        out_specs=[
            pl.BlockSpec((BLK, LATENT), lambda i: (i, 0)),
            pl.BlockSpec((BLK, LATENT), lambda i: (i, 0)),
        ],
        out_shape=[
            jax.ShapeDtypeStruct((N_NODES, LATENT), jnp.float32),
            jax.ShapeDtypeStruct((N_NODES, LATENT), jnp.float32),
        ],
    )(x, w)


def _upd_body(agg_ref, im_ref, w_ref, cur_ref):
    h = jnp.dot(agg_ref[...], w_ref[...], preferred_element_type=jnp.float32)
    cur_ref[...] = jnp.maximum(h + im_ref[...], 0.0)


def _upd(agg, im, w):
    return pl.pallas_call(
        _upd_body,
        grid=(N_BLK,),
        in_specs=[
            pl.BlockSpec((BLK, LATENT), lambda i: (i, 0)),
            pl.BlockSpec((BLK, LATENT), lambda i: (i, 0)),
            pl.BlockSpec((LATENT, LATENT), lambda i: (0, 0)),
        ],
        out_specs=pl.BlockSpec((BLK, LATENT), lambda i: (i, 0)),
        out_shape=jax.ShapeDtypeStruct((N_NODES, LATENT), jnp.float32),
    )(agg, im, w)


def _fin_body(agg_ref, im_ref, gid_ref, wc_ref, wo_ref, out_ref, pooled):
    i = pl.program_id(0)

    @pl.when(i == 0)
    def _():
        pooled[...] = jnp.zeros_like(pooled)

    h = jnp.dot(agg_ref[...], wc_ref[...], preferred_element_type=jnp.float32)
    cur = jnp.maximum(h + im_ref[...], 0.0)
    ids = gid_ref[0, 0, :]
    onehot = (ids[None, :] == lax.broadcasted_iota(jnp.int32, (N_GRAPHS, BLK), 0)
              ).astype(jnp.float32)
    pooled[...] += jnp.dot(onehot, cur, preferred_element_type=jnp.float32)

    @pl.when(i == N_BLK - 1)
    def _():
        out_ref[...] = jnp.maximum(
            jnp.dot(pooled[...], wo_ref[...], preferred_element_type=jnp.float32), 0.0)


def _fin(agg, im, gid, wc, wo):
    return pl.pallas_call(
        _fin_body,
        grid=(N_BLK,),
        in_specs=[
            pl.BlockSpec((BLK, LATENT), lambda i: (i, 0)),
            pl.BlockSpec((BLK, LATENT), lambda i: (i, 0)),
            pl.BlockSpec((1, 1, BLK), lambda i: (i, 0, 0)),
            pl.BlockSpec((LATENT, LATENT), lambda i: (0, 0)),
            pl.BlockSpec((LATENT, LATENT), lambda i: (0, 0)),
        ],
        out_specs=pl.BlockSpec((N_GRAPHS, LATENT), lambda i: (0, 0)),
        out_shape=jax.ShapeDtypeStruct((N_GRAPHS, LATENT), jnp.float32),
        scratch_shapes=[pltpu.VMEM((N_GRAPHS, LATENT), jnp.float32)],
    )(agg, im, gid, wc, wo)


@jax.jit
def _run(x, edge_index, graph_ids, W_n2l, W_conv, W_out):
    ei = edge_index.astype(jnp.int32)
    pad_src = jnp.zeros((NS, PAD_PER_SUB), jnp.int32)
    pad_dst = jnp.broadcast_to(
        JUNK + jnp.arange(PAD_PER_SUB, dtype=jnp.int32) % N_JUNK,
        (NS, PAD_PER_SUB))
    src = jnp.concatenate([(2 * ei[0]).reshape(NS, E_PER_SUB), pad_src],
                          axis=1).reshape(NS, N_CHUNKS, CHUNK)
    dst = jnp.concatenate([ei[1].reshape(NS, E_PER_SUB), pad_dst],
                          axis=1).reshape(NS, N_CHUNKS, CHUNK)
    gid = graph_ids.astype(jnp.int32).reshape(N_BLK, 1, BLK)
    im, cur = _proj(x, W_n2l)
    for _ in range(MAX_LV - 1):
        agg = _spmm(cur, src, dst)
        cur = _upd(agg, im, W_conv)
    agg = _spmm(cur, src, dst)
    return _fin(agg, im, gid, W_conv, W_out)


def kernel(x, edge_index, graph_ids, W_n2l, W_conv, W_out):
    return _run(x, edge_index, graph_ids, W_n2l, W_conv, W_out)


# speedup vs baseline: 3.3104x; 3.3104x over previous
import functools

import jax
import jax.numpy as jnp
from jax import lax
from jax.experimental import pallas as pl
from jax.experimental.pallas import tpu as pltpu
from jax.experimental.pallas import tpu_sc as plsc

LATENT = 128
N_NODES = 10000
N_EDGES = 320000
N_GRAPHS = 64
MAX_LV = 3

NC, NS = 2, 16
HALF = LATENT // NC
E_PER_SUB = N_EDGES // NS
CHUNK = 125
N_CHUNKS = E_PER_SUB // CHUNK
NBUF = 4
ROWS_PER_SUB = N_NODES // NS
ZROWS = 25

BLK = 2000
N_BLK = N_NODES // BLK


def _spmm_body(cur_hbm, src_hbm, dst_hbm, agg_hbm,
               agg_sh, src_v, dst_v, rows, zbuf, sems, isem):
    c = lax.axis_index("c")
    s = lax.axis_index("s")

    i0 = pltpu.async_copy(src_hbm.at[s], src_v, isem)
    i1 = pltpu.async_copy(dst_hbm.at[s], dst_v, isem)

    zero = jnp.zeros((16,), jnp.float32)
    for r in range(ZROWS):
        for q in range(HALF // 16):
            zbuf[r, pl.ds(q * 16, 16)] = zero
    base_row = s * ROWS_PER_SUB

    def zbody(i, carry):
        pltpu.sync_copy(zbuf, agg_sh.at[pl.ds(base_row + i * ZROWS, ZROWS)])
        return carry

    lax.fori_loop(0, ROWS_PER_SUB // ZROWS, zbody, 0)
    i0.wait()
    i1.wait()
    plsc.subcore_barrier()

    cur_c = cur_hbm.at[c]

    def start(j, q):
        pltpu.async_copy(cur_c.at[src_v.at[j]], rows[q], sems[q])

    def finish(j, q):
        pltpu.make_async_copy(cur_c.at[src_v.at[j]], rows[q], sems[q]).wait()
        pltpu.sync_copy(rows[q], agg_sh.at[dst_v.at[j]], add=True)

    for q in range(NBUF - 1):
        start(q, q)

    def body(t, carry):
        j0 = NBUF * t
        for q in range(NBUF):
            finish(j0 + q, q)
            start(j0 + q + NBUF - 1, (q + NBUF - 1) % NBUF)
        return carry

    lax.fori_loop(0, N_CHUNKS // NBUF - 1, body, 0)
    j0 = N_CHUNKS - NBUF
    finish(j0, 0)
    start(N_CHUNKS - 1, NBUF - 1)
    for q in range(1, NBUF):
        finish(j0 + q, q)

    plsc.subcore_barrier()
    pltpu.sync_copy(agg_sh.at[pl.ds(base_row, ROWS_PER_SUB)],
                    agg_hbm.at[pl.ds(base_row, ROWS_PER_SUB),
                               pl.ds(c * HALF, HALF)])


@functools.lru_cache(maxsize=1)
def _get_spmm():
    return pl.kernel(
        _spmm_body,
        out_type=jax.ShapeDtypeStruct((N_NODES, LATENT), jnp.float32),
        mesh=plsc.VectorSubcoreMesh(core_axis_name="c", subcore_axis_name="s",
                                    num_cores=NC, num_subcores=NS),
        scratch_types=[
            pltpu.VMEM_SHARED((N_NODES, HALF), jnp.float32),
            pltpu.VMEM((N_CHUNKS, CHUNK), jnp.int32),
            pltpu.VMEM((N_CHUNKS, CHUNK), jnp.int32),
            [pltpu.VMEM((CHUNK, HALF), jnp.float32) for _ in range(NBUF)],
            pltpu.VMEM((ZROWS, HALF), jnp.float32),
            [pltpu.SemaphoreType.DMA for _ in range(NBUF)],
            pltpu.SemaphoreType.DMA,
        ],
        compiler_params=pltpu.CompilerParams(use_tc_tiling_on_sc=False),
    )


def _spmm(cur2, src, dst):
    return _get_spmm()(cur2, src, dst)


def _split(cur_ref, h):
    cur_ref[0] = h[:, :HALF]
    cur_ref[1] = h[:, HALF:]


_CUR2_SPEC = pl.BlockSpec((NC, BLK, HALF), lambda i: (0, i, 0))
_CUR2_SHAPE = jax.ShapeDtypeStruct((NC, N_NODES, HALF), jnp.float32)


def _proj_body(x_ref, w_ref, im_ref, cur_ref):
    im = jnp.dot(x_ref[...], w_ref[...], preferred_element_type=jnp.float32)
    im_ref[...] = im
    _split(cur_ref, jnp.maximum(im, 0.0))


def _proj(x, w):
    return pl.pallas_call(
        _proj_body,
        grid=(N_BLK,),
        in_specs=[
            pl.BlockSpec((BLK, LATENT), lambda i: (i, 0)),
            pl.BlockSpec((LATENT, LATENT), lambda i: (0, 0)),
        ],
        out_specs=[
            pl.BlockSpec((BLK, LATENT), lambda i: (i, 0)),
            _CUR2_SPEC,
        ],
        out_shape=[
            jax.ShapeDtypeStruct((N_NODES, LATENT), jnp.float32),
            _CUR2_SHAPE,
        ],
    )(x, w)


def _upd_body(agg_ref, im_ref, w_ref, cur_ref):
    h = jnp.dot(agg_ref[...], w_ref[...], preferred_element_type=jnp.float32)
    _split(cur_ref, jnp.maximum(h + im_ref[...], 0.0))


def _upd(agg, im, w):
    return pl.pallas_call(
        _upd_body,
        grid=(N_BLK,),
        in_specs=[
            pl.BlockSpec((BLK, LATENT), lambda i: (i, 0)),
            pl.BlockSpec((BLK, LATENT), lambda i: (i, 0)),
            pl.BlockSpec((LATENT, LATENT), lambda i: (0, 0)),
        ],
        out_specs=_CUR2_SPEC,
        out_shape=_CUR2_SHAPE,
    )(agg, im, w)


def _fin_body(agg_ref, im_ref, gid_ref, wc_ref, wo_ref, out_ref, pooled):
    i = pl.program_id(0)

    @pl.when(i == 0)
    def _():
        pooled[...] = jnp.zeros_like(pooled)

    h = jnp.dot(agg_ref[...], wc_ref[...], preferred_element_type=jnp.float32)
    cur = jnp.maximum(h + im_ref[...], 0.0)
    ids = gid_ref[0, 0, :]
    onehot = (ids[None, :] == lax.broadcasted_iota(jnp.int32, (N_GRAPHS, BLK), 0)
              ).astype(jnp.float32)
    pooled[...] += jnp.dot(onehot, cur, preferred_element_type=jnp.float32)

    @pl.when(i == N_BLK - 1)
    def _():
        out_ref[...] = jnp.maximum(
            jnp.dot(pooled[...], wo_ref[...], preferred_element_type=jnp.float32), 0.0)


def _fin(agg, im, gid, wc, wo):
    return pl.pallas_call(
        _fin_body,
        grid=(N_BLK,),
        in_specs=[
            pl.BlockSpec((BLK, LATENT), lambda i: (i, 0)),
            pl.BlockSpec((BLK, LATENT), lambda i: (i, 0)),
            pl.BlockSpec((1, 1, BLK), lambda i: (i, 0, 0)),
            pl.BlockSpec((LATENT, LATENT), lambda i: (0, 0)),
            pl.BlockSpec((LATENT, LATENT), lambda i: (0, 0)),
        ],
        out_specs=pl.BlockSpec((N_GRAPHS, LATENT), lambda i: (0, 0)),
        out_shape=jax.ShapeDtypeStruct((N_GRAPHS, LATENT), jnp.float32),
        scratch_shapes=[pltpu.VMEM((N_GRAPHS, LATENT), jnp.float32)],
    )(agg, im, gid, wc, wo)


@jax.jit
def _run(x, edge_index, graph_ids, W_n2l, W_conv, W_out):
    ei = edge_index.astype(jnp.int32)
    src = ei[0].reshape(NS, N_CHUNKS, CHUNK)
    dst = ei[1].reshape(NS, N_CHUNKS, CHUNK)
    gid = graph_ids.astype(jnp.int32).reshape(N_BLK, 1, BLK)
    im, cur2 = _proj(x, W_n2l)
    for _ in range(MAX_LV - 1):
        agg = _spmm(cur2, src, dst)
        cur2 = _upd(agg, im, W_conv)
    agg = _spmm(cur2, src, dst)
    return _fin(agg, im, gid, W_conv, W_out)


def kernel(x, edge_index, graph_ids, W_n2l, W_conv, W_out):
    return _run(x, edge_index, graph_ids, W_n2l, W_conv, W_out)


# speedup vs baseline: 3.4237x; 1.0342x over previous
import functools

import jax
import jax.numpy as jnp
from jax import lax
from jax.experimental import pallas as pl
from jax.experimental.pallas import tpu as pltpu
from jax.experimental.pallas import tpu_sc as plsc

LATENT = 128
N_NODES = 10000
N_EDGES = 320000
N_GRAPHS = 64
MAX_LV = 3

NC, NS = 2, 16
HALF = LATENT // NC
E_PER_SUB = N_EDGES // NS
CHUNK = 80
N_CHUNKS = E_PER_SUB // CHUNK
NBUF = 5
ROWS_PER_SUB = N_NODES // NS
ZROWS = 25

BLK = 2000
N_BLK = N_NODES // BLK


def _spmm_body(cur_hbm, src_hbm, dst_hbm, agg_hbm,
               agg_sh, src_v, dst_v, rows, zbuf, sems, isem):
    c = lax.axis_index("c")
    s = lax.axis_index("s")

    i0 = pltpu.async_copy(src_hbm.at[pl.ds(s * E_PER_SUB, E_PER_SUB)], src_v, isem)
    i1 = pltpu.async_copy(dst_hbm.at[pl.ds(s * E_PER_SUB, E_PER_SUB)], dst_v, isem)

    zero = jnp.zeros((16,), jnp.float32)
    for r in range(ZROWS):
        for q in range(HALF // 16):
            zbuf[r, pl.ds(q * 16, 16)] = zero
    base_row = s * ROWS_PER_SUB

    def zbody(i, carry):
        pltpu.sync_copy(zbuf, agg_sh.at[pl.ds(base_row + i * ZROWS, ZROWS)])
        return carry

    lax.fori_loop(0, ROWS_PER_SUB // ZROWS, zbody, 0)
    i0.wait()
    i1.wait()
    plsc.subcore_barrier()

    cur_c = cur_hbm.at[c]

    def start(j, q):
        pltpu.async_copy(cur_c.at[src_v.at[pl.ds(j * CHUNK, CHUNK)]],
                         rows[q], sems[q])

    def finish(j, q):
        pltpu.make_async_copy(cur_c.at[src_v.at[pl.ds(j * CHUNK, CHUNK)]],
                              rows[q], sems[q]).wait()
        pltpu.sync_copy(rows[q], agg_sh.at[dst_v.at[pl.ds(j * CHUNK, CHUNK)]],
                        add=True)

    for q in range(NBUF - 1):
        start(q, q)

    def body(t, carry):
        j0 = NBUF * t
        for q in range(NBUF):
            finish(j0 + q, q)
            start(j0 + q + NBUF - 1, (q + NBUF - 1) % NBUF)
        return carry

    lax.fori_loop(0, N_CHUNKS // NBUF - 1, body, 0)
    j0 = N_CHUNKS - NBUF
    finish(j0, 0)
    start(N_CHUNKS - 1, NBUF - 1)
    for q in range(1, NBUF):
        finish(j0 + q, q)

    plsc.subcore_barrier()
    pltpu.sync_copy(agg_sh.at[pl.ds(base_row, ROWS_PER_SUB)],
                    agg_hbm.at[pl.ds(base_row, ROWS_PER_SUB),
                               pl.ds(c * HALF, HALF)])


@functools.lru_cache(maxsize=1)
def _get_spmm():
    return pl.kernel(
        _spmm_body,
        out_type=jax.ShapeDtypeStruct((N_NODES, LATENT), jnp.float32),
        mesh=plsc.VectorSubcoreMesh(core_axis_name="c", subcore_axis_name="s",
                                    num_cores=NC, num_subcores=NS),
        scratch_types=[
            pltpu.VMEM_SHARED((N_NODES, HALF), jnp.float32),
            pltpu.VMEM((E_PER_SUB,), jnp.int32),
            pltpu.VMEM((E_PER_SUB,), jnp.int32),
            [pltpu.VMEM((CHUNK, HALF), jnp.float32) for _ in range(NBUF)],
            pltpu.VMEM((ZROWS, HALF), jnp.float32),
            [pltpu.SemaphoreType.DMA for _ in range(NBUF)],
            pltpu.SemaphoreType.DMA,
        ],
        compiler_params=pltpu.CompilerParams(use_tc_tiling_on_sc=False),
    )


def _spmm(cur2, src, dst):
    return _get_spmm()(cur2, src, dst)


def _split(cur_ref, h):
    cur_ref[0] = h[:, :HALF]
    cur_ref[1] = h[:, HALF:]


_CUR2_SPEC = pl.BlockSpec((NC, BLK, HALF), lambda i: (0, i, 0))
_CUR2_SHAPE = jax.ShapeDtypeStruct((NC, N_NODES, HALF), jnp.float32)


def _proj_body(x_ref, w_ref, im_ref, cur_ref):
    im = jnp.dot(x_ref[...], w_ref[...], preferred_element_type=jnp.float32)
    im_ref[...] = im
    _split(cur_ref, jnp.maximum(im, 0.0))


def _proj(x, w):
    return pl.pallas_call(
        _proj_body,
        grid=(N_BLK,),
        in_specs=[
            pl.BlockSpec((BLK, LATENT), lambda i: (i, 0)),
            pl.BlockSpec((LATENT, LATENT), lambda i: (0, 0)),
        ],
        out_specs=[
            pl.BlockSpec((BLK, LATENT), lambda i: (i, 0)),
            _CUR2_SPEC,
        ],
        out_shape=[
            jax.ShapeDtypeStruct((N_NODES, LATENT), jnp.float32),
            _CUR2_SHAPE,
        ],
    )(x, w)


def _upd_body(agg_ref, im_ref, w_ref, cur_ref):
    h = jnp.dot(agg_ref[...], w_ref[...], preferred_element_type=jnp.float32)
    _split(cur_ref, jnp.maximum(h + im_ref[...], 0.0))


def _upd(agg, im, w):
    return pl.pallas_call(
        _upd_body,
        grid=(N_BLK,),
        in_specs=[
            pl.BlockSpec((BLK, LATENT), lambda i: (i, 0)),
            pl.BlockSpec((BLK, LATENT), lambda i: (i, 0)),
            pl.BlockSpec((LATENT, LATENT), lambda i: (0, 0)),
        ],
        out_specs=_CUR2_SPEC,
        out_shape=_CUR2_SHAPE,
    )(agg, im, w)


def _fin_body(agg_ref, im_ref, gid_ref, wc_ref, wo_ref, out_ref, pooled):
    i = pl.program_id(0)

    @pl.when(i == 0)
    def _():
        pooled[...] = jnp.zeros_like(pooled)

    h = jnp.dot(agg_ref[...], wc_ref[...], preferred_element_type=jnp.float32)
    cur = jnp.maximum(h + im_ref[...], 0.0)
    ids = gid_ref[0, 0, :]
    onehot = (ids[None, :] == lax.broadcasted_iota(jnp.int32, (N_GRAPHS, BLK), 0)
              ).astype(jnp.float32)
    pooled[...] += jnp.dot(onehot, cur, preferred_element_type=jnp.float32)

    @pl.when(i == N_BLK - 1)
    def _():
        out_ref[...] = jnp.maximum(
            jnp.dot(pooled[...], wo_ref[...], preferred_element_type=jnp.float32), 0.0)


def _fin(agg, im, gid, wc, wo):
    return pl.pallas_call(
        _fin_body,
        grid=(N_BLK,),
        in_specs=[
            pl.BlockSpec((BLK, LATENT), lambda i: (i, 0)),
            pl.BlockSpec((BLK, LATENT), lambda i: (i, 0)),
            pl.BlockSpec((1, 1, BLK), lambda i: (i, 0, 0)),
            pl.BlockSpec((LATENT, LATENT), lambda i: (0, 0)),
            pl.BlockSpec((LATENT, LATENT), lambda i: (0, 0)),
        ],
        out_specs=pl.BlockSpec((N_GRAPHS, LATENT), lambda i: (0, 0)),
        out_shape=jax.ShapeDtypeStruct((N_GRAPHS, LATENT), jnp.float32),
        scratch_shapes=[pltpu.VMEM((N_GRAPHS, LATENT), jnp.float32)],
    )(agg, im, gid, wc, wo)


@jax.jit
def _run(x, edge_index, graph_ids, W_n2l, W_conv, W_out):
    ei = edge_index.astype(jnp.int32)
    src = ei[0]
    dst = ei[1]
    gid = graph_ids.astype(jnp.int32).reshape(N_BLK, 1, BLK)
    im, cur2 = _proj(x, W_n2l)
    for _ in range(MAX_LV - 1):
        agg = _spmm(cur2, src, dst)
        cur2 = _upd(agg, im, W_conv)
    agg = _spmm(cur2, src, dst)
    return _fin(agg, im, gid, W_conv, W_out)


def kernel(x, edge_index, graph_ids, W_n2l, W_conv, W_out):
    return _run(x, edge_index, graph_ids, W_n2l, W_conv, W_out)
